# baseline (speedup 1.0000x reference)
import jax
import jax.numpy as jnp
from jax import lax
from jax.experimental import pallas as pl
from jax.experimental.pallas import tpu as pltpu
from jax.experimental.pallas import tpu_sc as plsc

N = 10000
E = 320000
D = 128
EPS = 1e-3

NC = 2
NS = 16
NW = NC * NS

CHUNK = 128
K0 = 160
K1 = 0
LP = 40
CH_TOT = NS * (K0 + K1)
E_PAD = CH_TOT * CHUNK
R = 10240
ROWS_PER_TILE = R // NS


def _sc_aggregate_body(src_hbm, dst_hbm, h_hbm, out_hbm,
                       idx_s, idx_d, rows, rows2, aggr_sh,
                       gsa, gsb, ssa, ssb):
    c = lax.axis_index("c")
    s = lax.axis_index("s")

    with jax.named_scope("agg_zero"):
        def zero_row(i, _):
            for j in range(D // 16):
                rows[i, pl.ds(j * 16, 16)] = jnp.zeros((16,), jnp.float32)
            return 0
        lax.fori_loop(0, CHUNK, zero_row, 0)

        def zero_copy(k, _):
            pltpu.sync_copy(
                rows, aggr_sh.at[pl.ds(s * ROWS_PER_TILE + k * CHUNK, CHUNK)])
            return 0
        lax.fori_loop(0, ROWS_PER_TILE // CHUNK, zero_copy, 0)
        plsc.subcore_barrier()

    def gather(j, buf, s):
        return pltpu.async_copy(h_hbm.at[idx_s.at[j]], buf, s)

    def scatter(j, buf, s):
        return pltpu.async_copy(buf, aggr_sh.at[idx_d.at[j]], s, add=True)

    def wait_gather(buf, s):
        pltpu.make_async_copy(h_hbm.at[idx_s.at[0]], buf, s).wait()

    def wait_scatter(buf, s):
        pltpu.make_async_copy(buf, aggr_sh.at[idx_d.at[0]], s).wait()

    wbase = s * K0
    nphases = jnp.where(c == 0, K0 // LP, K1 // LP)

    def phase(p, _):
        base = wbase + p * LP
        pltpu.sync_copy(src_hbm.at[pl.ds(base, LP)], idx_s)
        pltpu.sync_copy(dst_hbm.at[pl.ds(base, LP)], idx_d)

        gather(0, rows, gsa).wait()
        gather(1, rows2, gsb)
        scatter(0, rows, ssa)

        def step(t, _):
            wait_gather(rows2, gsb)
            wait_scatter(rows, ssa)
            gather(2 * t, rows, gsa)
            scatter(2 * t - 1, rows2, ssb)
            wait_gather(rows, gsa)
            wait_scatter(rows2, ssb)
            gather(2 * t + 1, rows2, gsb)
            scatter(2 * t, rows, ssa)
            return 0
        lax.fori_loop(1, LP // 2, step, 0)
        wait_gather(rows2, gsb)
        wait_scatter(rows, ssa)
        scatter(LP - 1, rows2, ssb).wait()
        return 0

    with jax.named_scope("agg_edges"):
        lax.fori_loop(0, nphases, phase, 0)
        plsc.subcore_barrier()

    with jax.named_scope("agg_writeback"):
        @pl.when(c == 0)
        def _():
            pltpu.sync_copy(aggr_sh.at[pl.ds(s * ROWS_PER_TILE, ROWS_PER_TILE)],
                            out_hbm.at[pl.ds(s * ROWS_PER_TILE, ROWS_PER_TILE)])


_SC_CACHE = {}


def _sc_aggregate(src2d, dst2d, h):
    if "fn" not in _SC_CACHE:
        _SC_CACHE["fn"] = pl.kernel(
            _sc_aggregate_body,
            out_type=jax.ShapeDtypeStruct((R, D), jnp.float32),
            mesh=plsc.VectorSubcoreMesh(core_axis_name="c",
                                        subcore_axis_name="s",
                                        num_cores=NC, num_subcores=NS),
            scratch_types=[
                pltpu.VMEM((LP, CHUNK), jnp.int32),
                pltpu.VMEM((LP, CHUNK), jnp.int32),
                pltpu.VMEM((CHUNK, D), jnp.float32),
                pltpu.VMEM((CHUNK, D), jnp.float32),
                pltpu.VMEM_SHARED((R, D), jnp.float32),
                pltpu.SemaphoreType.DMA,
                pltpu.SemaphoreType.DMA,
                pltpu.SemaphoreType.DMA,
                pltpu.SemaphoreType.DMA,
            ],
        )
    return _SC_CACHE["fn"](src2d, dst2d, h)


BM = 1000
NB = N // BM


def _mm_stats_body(a0_ref, h_ref, wrel_ref, brel_ref, wroot_ref,
                   y_ref, st_ref):
    a = a0_ref[...]
    h = h_ref[...]
    y = lax.dot_general(a, wrel_ref[...], (((1,), (1,)), ((), ())),
                        preferred_element_type=jnp.float32)
    y = y + lax.dot_general(h, wroot_ref[...], (((1,), (1,)), ((), ())),
                            preferred_element_type=jnp.float32)
    y = y + brel_ref[...]
    y_ref[...] = y
    s1 = jnp.sum(y)
    s2 = jnp.sum(y * y)
    col = lax.broadcasted_iota(jnp.int32, (8, 128), 1)
    st_ref[...] = jnp.where(col == 0, s1, jnp.where(col == 1, s2, 0.0))


def _norm_body(y_ref, st_ref, g_ref, b_ref, o_ref):
    st = st_ref[...]
    tot1 = jnp.sum(st[:, 0:1]) / 8.0
    tot2 = jnp.sum(st[:, 1:2]) / 8.0
    mean = tot1 / (N * D)
    var = tot2 / (N * D) - mean * mean
    inv = lax.rsqrt(var + EPS)
    g = g_ref[0, 0]
    b = b_ref[0, 0]
    o_ref[...] = jnp.maximum((y_ref[...] - mean) * (inv * g) + b, 0.0)


def _tc_layer(a0, h, W_rel, b_rel, W_root, gamma, beta):
    full = lambda shp: pl.BlockSpec(shp, lambda i: (0,) * len(shp))
    rows = pl.BlockSpec((BM, D), lambda i: (i, 0))
    y, st = pl.pallas_call(
        _mm_stats_body,
        grid=(NB,),
        in_specs=[rows, rows, full((D, D)), full((1, D)), full((D, D))],
        out_specs=[rows, pl.BlockSpec((8, 128), lambda i: (i, 0))],
        out_shape=[jax.ShapeDtypeStruct((N, D), jnp.float32),
                   jax.ShapeDtypeStruct((NB * 8, 128), jnp.float32)],
    )(a0, h, W_rel, b_rel.reshape(1, D), W_root)
    out = pl.pallas_call(
        _norm_body,
        grid=(NB,),
        in_specs=[rows, full((NB * 8, 128)), full((1, 1)), full((1, 1))],
        out_specs=rows,
        out_shape=jax.ShapeDtypeStruct((N, D), jnp.float32),
    )(y, st, gamma.reshape(1, 1), beta.reshape(1, 1))
    return out


def kernel(x, edge_index, W_rel0, b_rel0, W_root0, gamma0, beta0,
           W_rel1, b_rel1, W_root1, gamma1, beta1):
    pad = E_PAD - E
    src = jnp.concatenate([edge_index[0], jnp.zeros((pad,), jnp.int32)])
    dst = jnp.concatenate(
        [edge_index[1], N + (jnp.arange(pad, dtype=jnp.int32) % (R - N))])
    src2d = src.reshape(CH_TOT, CHUNK)
    dst2d = dst.reshape(CH_TOT, CHUNK)

    def layer(h, W_rel, b_rel, W_root, gamma, beta):
        agg = _sc_aggregate(src2d, dst2d, h)
        return _tc_layer(agg[0:N], h, W_rel, b_rel, W_root, gamma, beta)

    h = layer(x, W_rel0, b_rel0, W_root0, gamma0, beta0)
    h = layer(h, W_rel1, b_rel1, W_root1, gamma1, beta1)
    return h


# speedup vs baseline: 1.4102x; 1.4102x over previous
import jax
import jax.numpy as jnp
from jax import lax
from jax.experimental import pallas as pl
from jax.experimental.pallas import tpu as pltpu
from jax.experimental.pallas import tpu_sc as plsc

N = 10000
E = 320000
D = 128
EPS = 1e-3

NC = 2
NS = 16
NW = NC * NS

CHUNK = 128
K0 = 152
K1 = 8
LP = 8
CH_TOT = NS * (K0 + K1)
E_PAD = CH_TOT * CHUNK
R = 10240
ROWS_PER_TILE = R // NS


def _sc_aggregate_body(src_hbm, dst_hbm, h_hbm, out_hbm,
                       idx_s, idx_d, rows, rows2, aggr_sh,
                       gsa, gsb, ssa, ssb):
    c = lax.axis_index("c")
    s = lax.axis_index("s")

    with jax.named_scope("agg_zero"):
        def zero_row(i, _):
            for j in range(D // 16):
                rows[i, pl.ds(j * 16, 16)] = jnp.zeros((16,), jnp.float32)
            return 0
        lax.fori_loop(0, CHUNK, zero_row, 0)

        def zero_copy(k, _):
            pltpu.sync_copy(
                rows, aggr_sh.at[pl.ds(s * ROWS_PER_TILE + k * CHUNK, CHUNK)])
            return 0
        lax.fori_loop(0, ROWS_PER_TILE // CHUNK, zero_copy, 0)
        plsc.subcore_barrier()

    def gather(j, buf, sem):
        return pltpu.async_copy(h_hbm.at[idx_s.at[j]], buf, sem)

    def scatter(j, buf, sem):
        return pltpu.async_copy(buf, aggr_sh.at[idx_d.at[j]], sem, add=True)

    def wait_gather(buf, sem):
        pltpu.make_async_copy(h_hbm.at[idx_s.at[0]], buf, sem).wait()

    def wait_scatter(buf, sem):
        pltpu.make_async_copy(buf, aggr_sh.at[idx_d.at[0]], sem).wait()

    wbase = jnp.where(c == 0, s * K0, NS * K0 + s * K1)
    nphases = jnp.where(c == 0, K0 // LP, K1 // LP)

    def phase(p, _):
        base = wbase + p * LP
        pltpu.sync_copy(src_hbm.at[pl.ds(base, LP)], idx_s)
        pltpu.sync_copy(dst_hbm.at[pl.ds(base, LP)], idx_d)

        gather(0, rows, gsa).wait()
        gather(1, rows2, gsb)
        scatter(0, rows, ssa)

        def step(t, _):
            wait_gather(rows2, gsb)
            wait_scatter(rows, ssa)
            gather(2 * t, rows, gsa)
            scatter(2 * t - 1, rows2, ssb)
            wait_gather(rows, gsa)
            wait_scatter(rows2, ssb)
            gather(2 * t + 1, rows2, gsb)
            scatter(2 * t, rows, ssa)
            return 0
        lax.fori_loop(1, LP // 2, step, 0)
        wait_gather(rows2, gsb)
        wait_scatter(rows, ssa)
        scatter(LP - 1, rows2, ssb).wait()
        return 0

    with jax.named_scope("agg_edges"):
        lax.fori_loop(0, nphases, phase, 0)
        plsc.subcore_barrier()

    with jax.named_scope("agg_writeback"):
        pltpu.sync_copy(aggr_sh.at[pl.ds(s * ROWS_PER_TILE, ROWS_PER_TILE)],
                        out_hbm.at[pl.ds(c * R + s * ROWS_PER_TILE,
                                         ROWS_PER_TILE)])


_SC_CACHE = {}


def _sc_aggregate(src2d, dst2d, h):
    if "fn" not in _SC_CACHE:
        _SC_CACHE["fn"] = pl.kernel(
            _sc_aggregate_body,
            out_type=jax.ShapeDtypeStruct((2 * R, D), jnp.float32),
            mesh=plsc.VectorSubcoreMesh(core_axis_name="c",
                                        subcore_axis_name="s",
                                        num_cores=NC, num_subcores=NS),
            scratch_types=[
                pltpu.VMEM((LP, CHUNK), jnp.int32),
                pltpu.VMEM((LP, CHUNK), jnp.int32),
                pltpu.VMEM((CHUNK, D), jnp.float32),
                pltpu.VMEM((CHUNK, D), jnp.float32),
                pltpu.VMEM_SHARED((R, D), jnp.float32),
                pltpu.SemaphoreType.DMA,
                pltpu.SemaphoreType.DMA,
                pltpu.SemaphoreType.DMA,
                pltpu.SemaphoreType.DMA,
            ],
        )
    return _SC_CACHE["fn"](src2d, dst2d, h)


BM = 1000
NB = N // BM


def _mm_stats_body(a0_ref, a1_ref, h_ref, wrel_ref, brel_ref, wroot_ref,
                   y_ref, st_ref):
    a = a0_ref[...] + a1_ref[...]
    h = h_ref[...]
    y = lax.dot_general(a, wrel_ref[...], (((1,), (1,)), ((), ())),
                        preferred_element_type=jnp.float32)
    y = y + lax.dot_general(h, wroot_ref[...], (((1,), (1,)), ((), ())),
                            preferred_element_type=jnp.float32)
    y = y + brel_ref[...]
    y_ref[...] = y
    s1 = jnp.sum(y)
    s2 = jnp.sum(y * y)
    col = lax.broadcasted_iota(jnp.int32, (8, 128), 1)
    st_ref[...] = jnp.where(col == 0, s1, jnp.where(col == 1, s2, 0.0))


def _norm_body(y_ref, st_ref, g_ref, b_ref, o_ref):
    st = st_ref[...]
    tot1 = jnp.sum(st[:, 0:1]) / 8.0
    tot2 = jnp.sum(st[:, 1:2]) / 8.0
    mean = tot1 / (N * D)
    var = tot2 / (N * D) - mean * mean
    inv = lax.rsqrt(var + EPS)
    g = g_ref[0, 0]
    b = b_ref[0, 0]
    o_ref[...] = jnp.maximum((y_ref[...] - mean) * (inv * g) + b, 0.0)


def _tc_layer(a0, a1, h, W_rel, b_rel, W_root, gamma, beta):
    full = lambda shp: pl.BlockSpec(shp, lambda i: (0,) * len(shp))
    rows = pl.BlockSpec((BM, D), lambda i: (i, 0))
    y, st = pl.pallas_call(
        _mm_stats_body,
        grid=(NB,),
        in_specs=[rows, rows, rows, full((D, D)), full((1, D)), full((D, D))],
        out_specs=[rows, pl.BlockSpec((8, 128), lambda i: (i, 0))],
        out_shape=[jax.ShapeDtypeStruct((N, D), jnp.float32),
                   jax.ShapeDtypeStruct((NB * 8, 128), jnp.float32)],
    )(a0, a1, h, W_rel, b_rel.reshape(1, D), W_root)
    out = pl.pallas_call(
        _norm_body,
        grid=(NB,),
        in_specs=[rows, full((NB * 8, 128)), full((1, 1)), full((1, 1))],
        out_specs=rows,
        out_shape=jax.ShapeDtypeStruct((N, D), jnp.float32),
    )(y, st, gamma.reshape(1, 1), beta.reshape(1, 1))
    return out


def kernel(x, edge_index, W_rel0, b_rel0, W_root0, gamma0, beta0,
           W_rel1, b_rel1, W_root1, gamma1, beta1):
    pad = E_PAD - E
    src = jnp.concatenate([edge_index[0], jnp.zeros((pad,), jnp.int32)])
    dst = jnp.concatenate(
        [edge_index[1], N + (jnp.arange(pad, dtype=jnp.int32) % (R - N))])
    src2d = src.reshape(CH_TOT, CHUNK)
    dst2d = dst.reshape(CH_TOT, CHUNK)

    def layer(h, W_rel, b_rel, W_root, gamma, beta):
        agg = _sc_aggregate(src2d, dst2d, h)
        return _tc_layer(agg[0:N], agg[R:R + N], h,
                         W_rel, b_rel, W_root, gamma, beta)

    h = layer(x, W_rel0, b_rel0, W_root0, gamma0, beta0)
    h = layer(h, W_rel1, b_rel1, W_root1, gamma1, beta1)
    return h
